# baseline (speedup 1.0000x reference)
import jax
import jax.numpy as jnp
from jax import lax
from jax.experimental import pallas as pl
from jax.experimental.pallas import tpu as pltpu
from jax.experimental.pallas import tpu_sc as plsc

D_MODEL = 1024
SCALE = 32.0
LANES = 16

NC, NS = 2, 16
NW = NC * NS
B_TOT = 4 * 4096
B_PER_W = B_TOT // NW
CHUNK = 16
NBUF = 4
NCHUNK = B_PER_W // CHUNK
VECS = CHUNK * D_MODEL // LANES
XCOLS = 4096
W_PER_XROW = XCOLS // B_PER_W


def _emb_body(idx_hbm, lut_hbm, out_hbm, idx_v, *scratch):
    bufs = scratch[:NBUF]
    gsems = scratch[NBUF:2 * NBUF]
    ssems = scratch[2 * NBUF:3 * NBUF]
    wid = lax.axis_index("s") * NC + lax.axis_index("c")
    base = wid * B_PER_W

    pltpu.sync_copy(
        idx_hbm.at[wid // W_PER_XROW, pl.ds((wid % W_PER_XROW) * B_PER_W, B_PER_W)],
        idx_v)

    def gather_desc(i, b):
        return pltpu.make_async_copy(
            lut_hbm.at[idx_v.at[pl.ds(i * CHUNK, CHUNK)]], bufs[b], gsems[b])

    def store_desc(i, b):
        return pltpu.make_async_copy(
            bufs[b], out_hbm.at[pl.ds(base + i * CHUNK, CHUNK)], ssems[b])

    def scale(b):
        buf = bufs[b]

        @plsc.parallel_loop(0, VECS, unroll=8)
        def _s(j):
            r = lax.shift_right_logical(j, 6)
            c = (j & 63) * LANES
            buf[r, pl.ds(c, LANES)] = buf[r, pl.ds(c, LANES)] * SCALE

    def step(i, b):
        gather_desc(i, b).wait()
        scale(b)
        store_desc(i, b).start()
        j = i + NBUF - 1
        pb = (b - 1) % NBUF

        @pl.when(jnp.logical_and(i >= 1, j < NCHUNK))
        def _():
            store_desc(i - 1, pb).wait()

        @pl.when(j < NCHUNK)
        def _():
            gather_desc(j, pb).start()

    for b in range(NBUF - 1):
        gather_desc(b, b).start()

    @pl.loop(0, NCHUNK, step=NBUF)
    def _group(g):
        for b in range(NBUF):
            step(g + b, b)

    for b in range(NBUF):
        store_desc(NCHUNK - NBUF + b, b).wait()


def kernel(x, lut):
    idx = x.astype(jnp.int32)
    out = pl.kernel(
        _emb_body,
        out_type=jax.ShapeDtypeStruct((B_TOT, D_MODEL), jnp.float32),
        mesh=plsc.VectorSubcoreMesh(core_axis_name="c", subcore_axis_name="s"),
        scratch_types=[
            pltpu.VMEM((B_PER_W,), jnp.int32),
        ] + [pltpu.VMEM((CHUNK, D_MODEL), jnp.float32) for _ in range(NBUF)]
          + [pltpu.SemaphoreType.DMA for _ in range(2 * NBUF)],
    )(idx, lut)
    return out.reshape(x.shape + (D_MODEL,))


# speedup vs baseline: 1.0161x; 1.0161x over previous
import jax
import jax.numpy as jnp
from jax import lax
from jax.experimental import pallas as pl
from jax.experimental.pallas import tpu as pltpu
from jax.experimental.pallas import tpu_sc as plsc

D_MODEL = 1024
SCALE = 32.0
LANES = 16

NC, NS = 2, 16
NW = NC * NS
B_TOT = 4 * 4096
B_PER_W = B_TOT // NW
CHUNK = 16
NBUF = 6
NCHUNK = B_PER_W // CHUNK
VECS = CHUNK * D_MODEL // LANES
XCOLS = 4096
W_PER_XROW = XCOLS // B_PER_W


def _emb_body(idx_hbm, lut_hbm, out_hbm, idx_v, *scratch):
    bufs = scratch[:NBUF]
    gsems = scratch[NBUF:2 * NBUF]
    ssems = scratch[2 * NBUF:3 * NBUF]
    wid = lax.axis_index("s") * NC + lax.axis_index("c")
    base = wid * B_PER_W

    pltpu.sync_copy(
        idx_hbm.at[wid // W_PER_XROW, pl.ds((wid % W_PER_XROW) * B_PER_W, B_PER_W)],
        idx_v)

    def gather_desc(i, b):
        return pltpu.make_async_copy(
            lut_hbm.at[idx_v.at[pl.ds(i * CHUNK, CHUNK)]], bufs[b], gsems[b])

    def store_desc(i, b):
        return pltpu.make_async_copy(
            bufs[b], out_hbm.at[pl.ds(base + i * CHUNK, CHUNK)], ssems[b])

    def scale(b):
        buf = bufs[b]

        @plsc.parallel_loop(0, VECS, unroll=8)
        def _s(j):
            r = lax.shift_right_logical(j, 6)
            c = (j & 63) * LANES
            buf[r, pl.ds(c, LANES)] = buf[r, pl.ds(c, LANES)] * SCALE

    def step(i, b):
        gather_desc(i, b).wait()
        scale(b)
        store_desc(i, b).start()
        j = i + NBUF - 1
        pb = (b - 1) % NBUF

        @pl.when(jnp.logical_and(i >= 1, j < NCHUNK))
        def _():
            store_desc(i - 1, pb).wait()

        @pl.when(j < NCHUNK)
        def _():
            gather_desc(j, pb).start()

    for b in range(NBUF - 1):
        gather_desc(b, b).start()

    @pl.loop(0, NCHUNK, step=NBUF)
    def _group(g):
        for b in range(NBUF):
            i = g + b
            if NCHUNK % NBUF == 0:
                step(i, b)
            else:
                @pl.when(i < NCHUNK)
                def _():
                    step(i, b)

    for k in range(NBUF):
        i = NCHUNK - NBUF + k
        store_desc(i, i % NBUF).wait()


def kernel(x, lut):
    idx = x.astype(jnp.int32)
    out = pl.kernel(
        _emb_body,
        out_type=jax.ShapeDtypeStruct((B_TOT, D_MODEL), jnp.float32),
        mesh=plsc.VectorSubcoreMesh(core_axis_name="c", subcore_axis_name="s"),
        scratch_types=[
            pltpu.VMEM((B_PER_W,), jnp.int32),
        ] + [pltpu.VMEM((CHUNK, D_MODEL), jnp.float32) for _ in range(NBUF)]
          + [pltpu.SemaphoreType.DMA for _ in range(2 * NBUF)],
    )(idx, lut)
    return out.reshape(x.shape + (D_MODEL,))
